# Initial kernel scaffold; baseline (speedup 1.0000x reference)
import jax
import jax.numpy as jnp
from jax.experimental import pallas as pl


def kernel(x_person, x_movie, e_ai_src, e_ai_dst, e_di_src, e_di_dst, e_ww_src, e_ww_dst, Wl0_ai, bl0_ai, Wr0_ai, Wl0_di, bl0_di, Wr0_di, Wl0_ww, bl0_ww, Wr0_ww, Wl1_ai, bl1_ai, Wr1_ai, Wl1_di, bl1_di, Wr1_di, Wl1_ww, bl1_ww, Wr1_ww, W_out, b_out):
    raise NotImplementedError("write your pallas kernel here")


# speedup vs baseline: 4.2161x; 4.2161x over previous
import functools

import jax
import jax.numpy as jnp
from jax import lax
from jax.experimental import pallas as pl
from jax.experimental.pallas import tpu as pltpu
from jax.experimental.pallas import tpu_sc as plsc

F32 = jnp.float32
N_PERSON = 50000
N_MOVIE = 10000
HID = 64
NC = 2
NS = 16
NW = NC * NS
CHUNK_ROWS = 8
CHUNK = CHUNK_ROWS * 128
HALF = N_PERSON // NC
M_ACC = 10112
WQ = HALF // 2
WQ_ACC = 12544
W_CNT = 25088

_SC_PARAMS = pltpu.CompilerParams(use_tc_tiling_on_sc=False)


def _nt(total_chunks, nworkers, wid):
  base, extra = divmod(total_chunks, nworkers)
  return jnp.where(wid < extra, base + 1, base)


def _idx_scratch(n):
  return [pltpu.VMEM((128,), jnp.int32) for _ in range(n)]


def _load_idx(e, j, idl, sem):
  descs = [
      pltpu.async_copy(e.at[pl.ds(j * CHUNK + g * 128, 128)], idl[g], sem)
      for g in range(CHUNK_ROWS)
  ]
  for d in descs:
    d.wait()


def _remap(idl, lo, rlen):
  for g in range(CHUNK_ROWS):
    def kbody(k, _, g=g):
      dv = idl[g][pl.ds(k * 16, 16)]
      inr = (dv >= lo) & (dv < lo + rlen)
      idl[g][pl.ds(k * 16, 16)] = jnp.where(inr, dv - lo, rlen)
      return 0
    lax.fori_loop(0, 128 // 16, kbody, 0)


def _sum_rel(es, ed, h, acc, isl, idl, rows, sem, nt, first, stride,
             lo=None, rlen=None):
  def tbody(t, _):
    j = first + t * stride
    _load_idx(es, j, isl, sem)
    _load_idx(ed, j, idl, sem)
    gds = [
        pltpu.async_copy(h.at[isl[g]], rows.at[pl.ds(g * 128, 128)], sem)
        for g in range(CHUNK_ROWS)
    ]
    if lo is not None:
      _remap(idl, lo, rlen)
    for d in gds:
      d.wait()
    for g in range(CHUNK_ROWS):
      pltpu.sync_copy(rows.at[pl.ds(g * 128, 128)], acc.at[idl[g]], add=True)
    return 0

  lax.fori_loop(0, nt, tbody, 0)


def _count_rel(ed, csh, idl, obuf, sem, nt, first, stride, lo=None,
               rlen=None):
  def tbody(t, _):
    j = first + t * stride
    _load_idx(ed, j, idl, sem)
    if lo is not None:
      _remap(idl, lo, rlen)
    for g in range(CHUNK_ROWS):
      pltpu.sync_copy(obuf, csh.at[idl[g]], add=True)
    return 0

  lax.fori_loop(0, nt, tbody, 0)


def _sc_counts(ed_ai, ed_di, ed_ww, z16, o16):
  chunks_ai = ed_ai.shape[0] // CHUNK
  chunks_di = ed_di.shape[0] // CHUNK
  chunks_ww = ed_ww.shape[0] // CHUNK

  mesh = plsc.VectorSubcoreMesh(core_axis_name="c", subcore_axis_name="s")

  @functools.partial(
      pl.kernel,
      out_type=(jax.ShapeDtypeStruct((NC, N_MOVIE, 16), F32),
                jax.ShapeDtypeStruct((NC, N_MOVIE, 16), F32),
                jax.ShapeDtypeStruct((NC, W_CNT, 16), F32)),
      mesh=mesh,
      compiler_params=_SC_PARAMS,
      scratch_types=_idx_scratch(CHUNK_ROWS) + [
          pltpu.VMEM((128, 16), F32),
          pltpu.VMEM_SHARED((M_ACC, 16), F32),
          pltpu.VMEM_SHARED((M_ACC, 16), F32),
          pltpu.VMEM_SHARED((W_CNT, 16), F32),
          pltpu.SemaphoreType.DMA,
      ])
  def body(ed_ai_r, ed_di_r, ed_ww_r, z16_r, o16_r, c_ai_r, c_di_r, c_ww_r,
           *rest):
    idl = list(rest[:CHUNK_ROWS])
    obuf, csh_a, csh_d, csh_w, sem = rest[CHUNK_ROWS:]
    c = lax.axis_index("c")
    s = lax.axis_index("s")
    w = c * NS + s

    pltpu.sync_copy(o16_r, obuf)
    rpt = M_ACC // NS
    pltpu.sync_copy(z16_r.at[pl.ds(0, rpt)], csh_a.at[pl.ds(s * rpt, rpt)])
    pltpu.sync_copy(z16_r.at[pl.ds(0, rpt)], csh_d.at[pl.ds(s * rpt, rpt)])
    wpt = W_CNT // NS
    pltpu.sync_copy(z16_r.at[pl.ds(0, 1024)],
                    csh_w.at[pl.ds(s * wpt, 1024)])
    pltpu.sync_copy(z16_r.at[pl.ds(0, wpt - 1024)],
                    csh_w.at[pl.ds(s * wpt + 1024, wpt - 1024)])
    plsc.subcore_barrier()

    _count_rel(ed_ai_r, csh_a, idl, obuf, sem, _nt(chunks_ai, NW, w), w, NW)
    _count_rel(ed_di_r, csh_d, idl, obuf, sem, _nt(chunks_di, NW, w), w, NW)
    _count_rel(ed_ww_r, csh_w, idl, obuf, sem, _nt(chunks_ww, NS, s), s, NS,
               lo=c * HALF, rlen=HALF)

    plsc.subcore_barrier()
    @pl.when(s < N_MOVIE // 1000)
    def _():
      pltpu.sync_copy(csh_a.at[pl.ds(s * 1000, 1000)],
                      c_ai_r.at[c, pl.ds(s * 1000, 1000)])
      pltpu.sync_copy(csh_d.at[pl.ds(s * 1000, 1000)],
                      c_di_r.at[c, pl.ds(s * 1000, 1000)])
    pltpu.sync_copy(csh_w.at[pl.ds(s * wpt, wpt)],
                    c_ww_r.at[c, pl.ds(s * wpt, wpt)])

  return body(ed_ai, ed_di, ed_ww, z16, o16)


def _sc_rel(h, es, ed, z64):
  chunks = es.shape[0] // CHUNK

  mesh = plsc.VectorSubcoreMesh(core_axis_name="c", subcore_axis_name="s")

  @functools.partial(
      pl.kernel,
      out_type=jax.ShapeDtypeStruct((NC, N_MOVIE, HID), F32),
      mesh=mesh,
      compiler_params=_SC_PARAMS,
      scratch_types=_idx_scratch(2 * CHUNK_ROWS) + [
          pltpu.VMEM((CHUNK, HID), F32),
          pltpu.VMEM_SHARED((M_ACC, HID), F32),
          pltpu.SemaphoreType.DMA,
      ])
  def body(h_r, es_r, ed_r, z64_r, s_r, *rest):
    isl = list(rest[:CHUNK_ROWS])
    idl = list(rest[CHUNK_ROWS:2 * CHUNK_ROWS])
    rows, acc, sem = rest[2 * CHUNK_ROWS:]
    c = lax.axis_index("c")
    s = lax.axis_index("s")
    w = c * NS + s

    rpt = M_ACC // NS
    pltpu.sync_copy(z64_r.at[pl.ds(0, rpt)], acc.at[pl.ds(s * rpt, rpt)])
    plsc.subcore_barrier()

    _sum_rel(es_r, ed_r, h_r, acc, isl, idl, rows, sem,
             _nt(chunks, NW, w), w, NW)

    plsc.subcore_barrier()
    @pl.when(s < N_MOVIE // 1000)
    def _():
      pltpu.sync_copy(acc.at[pl.ds(s * 1000, 1000)],
                      s_r.at[c, pl.ds(s * 1000, 1000)])

  return body(h, es, ed, z64)


def _sc_ww_range(h, es, ed, z64, base):
  chunks = es.shape[0] // CHUNK

  mesh = plsc.VectorSubcoreMesh(core_axis_name="c", subcore_axis_name="s")

  @functools.partial(
      pl.kernel,
      out_type=jax.ShapeDtypeStruct((NC, WQ_ACC, HID), F32),
      mesh=mesh,
      compiler_params=_SC_PARAMS,
      scratch_types=_idx_scratch(2 * CHUNK_ROWS) + [
          pltpu.VMEM((CHUNK, HID), F32),
          pltpu.VMEM_SHARED((WQ_ACC, HID), F32),
          pltpu.SemaphoreType.DMA,
      ])
  def body(h_r, es_r, ed_r, z64_r, s_r, *rest):
    isl = list(rest[:CHUNK_ROWS])
    idl = list(rest[CHUNK_ROWS:2 * CHUNK_ROWS])
    rows, acc, sem = rest[2 * CHUNK_ROWS:]
    c = lax.axis_index("c")
    s = lax.axis_index("s")
    lo = c * HALF + base

    rpt = WQ_ACC // NS
    pltpu.sync_copy(z64_r.at[pl.ds(0, rpt)], acc.at[pl.ds(s * rpt, rpt)])
    plsc.subcore_barrier()

    _sum_rel(es_r, ed_r, h_r, acc, isl, idl, rows, sem,
             _nt(chunks, NS, s), s, NS, lo=lo, rlen=WQ)

    plsc.subcore_barrier()
    pltpu.sync_copy(acc.at[pl.ds(s * rpt, rpt)],
                    s_r.at[c, pl.ds(s * rpt, rpt)])

  return body(h, es, ed, z64)


def _tc_proj0(x_person, Wl0_ai, Wl0_di, Wl0_ww, Wr0_ww):
  R = 1000
  grid = (N_PERSON // R,)
  D = x_person.shape[1]

  def body(x_r, wa_r, wd_r, ww_r, wr_r, ha_r, hd_r, hw_r, xr_r):
    x = x_r[...]
    ha_r[...] = jnp.dot(x, wa_r[...], preferred_element_type=F32)
    hd_r[...] = jnp.dot(x, wd_r[...], preferred_element_type=F32)
    hw_r[...] = jnp.dot(x, ww_r[...], preferred_element_type=F32)
    xr_r[...] = jnp.dot(x, wr_r[...], preferred_element_type=F32)

  wspec = pl.BlockSpec((D, HID), lambda i: (0, 0))
  ospec = pl.BlockSpec((R, HID), lambda i: (i, 0))
  return pl.pallas_call(
      body,
      grid=grid,
      in_specs=[pl.BlockSpec((R, D), lambda i: (i, 0))] + [wspec] * 4,
      out_specs=[ospec] * 4,
      out_shape=[jax.ShapeDtypeStruct((N_PERSON, HID), F32)] * 4,
  )(x_person, Wl0_ai, Wl0_di, Wl0_ww, Wr0_ww)


def _tc_movie_combine(S_ai, c_ai, S_di, c_di, x_movie, Wr0_ai, Wr0_di,
                      bl0_ai, bl0_di):
  R = 1000
  D = x_movie.shape[1]

  def body(sa_r, ca_r, sd_r, cd_r, x_r, wa_r, wd_r, ba_r, bd_r,
           mv_r, cma_r, cmd_r):
    ca = jnp.maximum(ca_r[0] + ca_r[1], 1.0)
    cd = jnp.maximum(cd_r[0] + cd_r[1], 1.0)
    sa = (sa_r[0] + sa_r[1]) / ca
    sd = (sd_r[0] + sd_r[1]) / cd
    xr = jnp.dot(x_r[...], wa_r[...] + wd_r[...], preferred_element_type=F32)
    mv_r[...] = jnp.maximum(sa + sd + xr + ba_r[...] + bd_r[...], 0.0)
    cma_r[...] = ca
    cmd_r[...] = cd

  sspec = pl.BlockSpec((NC, R, HID), lambda i: (0, i, 0))
  cspec = pl.BlockSpec((NC, R, 1), lambda i: (0, i, 0))
  wspec = pl.BlockSpec((D, HID), lambda i: (0, 0))
  bspec = pl.BlockSpec((HID,), lambda i: (0,))
  return pl.pallas_call(
      body,
      grid=(N_MOVIE // R,),
      in_specs=[sspec, cspec, sspec, cspec,
                pl.BlockSpec((R, D), lambda i: (i, 0)), wspec, wspec,
                bspec, bspec],
      out_specs=[pl.BlockSpec((R, HID), lambda i: (i, 0)),
                 pl.BlockSpec((R, 1), lambda i: (i, 0)),
                 pl.BlockSpec((R, 1), lambda i: (i, 0))],
      out_shape=[jax.ShapeDtypeStruct((N_MOVIE, HID), F32),
                 jax.ShapeDtypeStruct((N_MOVIE, 1), F32),
                 jax.ShapeDtypeStruct((N_MOVIE, 1), F32)],
  )(S_ai, c_ai, S_di, c_di, x_movie, Wr0_ai, Wr0_di, bl0_ai, bl0_di)


def _tc_pe_proj1(S_ww, cw_col, xr_ww, bl0_ww, Wl1_ai, Wl1_di):
  R = 1000
  grid = (N_PERSON // R,)

  def body(sw_r, cw_r, xr_r, b_r, wa_r, wd_r, ga_r, gd_r):
    cw = jnp.maximum(cw_r[...], 1.0)
    pe = jnp.maximum(sw_r[...] / cw + xr_r[...] + b_r[...], 0.0)
    ga_r[...] = jnp.dot(pe, wa_r[...], preferred_element_type=F32)
    gd_r[...] = jnp.dot(pe, wd_r[...], preferred_element_type=F32)

  ospec = pl.BlockSpec((R, HID), lambda i: (i, 0))
  return pl.pallas_call(
      body,
      grid=grid,
      in_specs=[
          pl.BlockSpec((R, HID), lambda i: (i, 0)),
          pl.BlockSpec((R, 1), lambda i: (i, 0)),
          pl.BlockSpec((R, HID), lambda i: (i, 0)),
          pl.BlockSpec((HID,), lambda i: (0,)),
          pl.BlockSpec((HID, HID), lambda i: (0, 0)),
          pl.BlockSpec((HID, HID), lambda i: (0, 0)),
      ],
      out_specs=[ospec, ospec],
      out_shape=[jax.ShapeDtypeStruct((N_PERSON, HID), F32)] * 2,
  )(S_ww, cw_col, xr_ww, bl0_ww, Wl1_ai, Wl1_di)


def _tc_head(S1_ai, S1_di, cma, cmd, mv, Wr1_ai, Wr1_di, bl1_ai, bl1_di,
             w_out_row, b_out):
  R = 1000

  def body(sa_r, sd_r, ca_r, cd_r, mv_r, wa_r, wd_r, ba_r, bd_r, wo_r, bo_r,
           out_r):
    xr = jnp.dot(mv_r[...], wa_r[...] + wd_r[...], preferred_element_type=F32)
    mv2 = jnp.maximum(
        (sa_r[0] + sa_r[1]) / ca_r[...] + (sd_r[0] + sd_r[1]) / cd_r[...]
        + xr + ba_r[...] + bd_r[...], 0.0)
    out_r[...] = jnp.dot(mv2, wo_r[...], preferred_element_type=F32) + bo_r[0]

  sspec = pl.BlockSpec((NC, R, HID), lambda i: (0, i, 0))
  cspec = pl.BlockSpec((R, 1), lambda i: (i, 0))
  wspec = pl.BlockSpec((HID, HID), lambda i: (0, 0))
  bspec = pl.BlockSpec((HID,), lambda i: (0,))
  return pl.pallas_call(
      body,
      grid=(N_MOVIE // R,),
      in_specs=[sspec, sspec, cspec, cspec,
                pl.BlockSpec((R, HID), lambda i: (i, 0)), wspec, wspec,
                bspec, bspec,
                pl.BlockSpec((HID, 1), lambda i: (0, 0)),
                pl.BlockSpec((1,), lambda i: (0,))],
      out_specs=pl.BlockSpec((R, 1), lambda i: (i, 0)),
      out_shape=jax.ShapeDtypeStruct((N_MOVIE, 1), F32),
  )(S1_ai, S1_di, cma, cmd, mv, Wr1_ai, Wr1_di, bl1_ai, bl1_di,
    w_out_row, b_out)


def kernel(x_person, x_movie, e_ai_src, e_ai_dst, e_di_src, e_di_dst,
           e_ww_src, e_ww_dst, Wl0_ai, bl0_ai, Wr0_ai, Wl0_di, bl0_di, Wr0_di,
           Wl0_ww, bl0_ww, Wr0_ww, Wl1_ai, bl1_ai, Wr1_ai, Wl1_di, bl1_di,
           Wr1_di, Wl1_ww, bl1_ww, Wr1_ww, W_out, b_out):
  def _prep(src, dst, pad_dst):
    n = src.shape[0]
    m = -(-n // CHUNK) * CHUNK
    es = jnp.pad(src.astype(jnp.int32), (0, m - n))
    ed = jnp.pad(dst.astype(jnp.int32), (0, m - n), constant_values=pad_dst)
    return es, ed

  es_ai, ed_ai = _prep(e_ai_src, e_ai_dst, N_MOVIE)
  es_di, ed_di = _prep(e_di_src, e_di_dst, N_MOVIE)
  es_ww, ed_ww = _prep(e_ww_src, e_ww_dst, -1)

  z64 = jnp.zeros((1024, HID), F32)
  z16 = jnp.zeros((1024, 16), F32)
  o16 = jnp.ones((128, 16), F32)

  c_ai_raw, c_di_raw, c_ww_raw = _sc_counts(ed_ai, ed_di, ed_ww, z16, o16)

  h_ai, h_di, h_ww, xr_ww = _tc_proj0(x_person, Wl0_ai, Wl0_di, Wl0_ww, Wr0_ww)

  S_ai = _sc_rel(h_ai, es_ai, ed_ai, z64)
  S_di = _sc_rel(h_di, es_di, ed_di, z64)
  Sw0 = _sc_ww_range(h_ww, es_ww, ed_ww, z64, 0)
  Sw1 = _sc_ww_range(h_ww, es_ww, ed_ww, z64, WQ)
  S_ww = jnp.concatenate(
      [Sw0[0, :WQ], Sw1[0, :WQ], Sw0[1, :WQ], Sw1[1, :WQ]], axis=0)

  c_ai = c_ai_raw[:, :, :1]
  c_di = c_di_raw[:, :, :1]
  cw_col = jnp.concatenate([c_ww_raw[0, :HALF], c_ww_raw[1, :HALF]],
                           axis=0)[:, :1]

  mv, cma, cmd = _tc_movie_combine(S_ai, c_ai, S_di, c_di, x_movie,
                                   Wr0_ai, Wr0_di, bl0_ai, bl0_di)
  g_ai, g_di = _tc_pe_proj1(S_ww, cw_col, xr_ww, bl0_ww, Wl1_ai, Wl1_di)

  S1_ai = _sc_rel(g_ai, es_ai, ed_ai, z64)
  S1_di = _sc_rel(g_di, es_di, ed_di, z64)

  out = _tc_head(S1_ai, S1_di, cma, cmd, mv, Wr1_ai, Wr1_di, bl1_ai, bl1_di,
                 W_out, b_out)
  return out.reshape(N_MOVIE)


# speedup vs baseline: 4.5716x; 1.0843x over previous
import functools

import jax
import jax.numpy as jnp
from jax import lax
from jax.experimental import pallas as pl
from jax.experimental.pallas import tpu as pltpu
from jax.experimental.pallas import tpu_sc as plsc

F32 = jnp.float32
N_PERSON = 50000
N_MOVIE = 10000
HID = 64
NC = 2
NS = 16
NW = NC * NS
CHUNK_ROWS = 4
CHUNK = CHUNK_ROWS * 128
HALF = N_PERSON // NC
M_ACC = 10112
WQ = HALF // 2
WQ_ACC = 12544
W_CNT = 25088

_SC_PARAMS = pltpu.CompilerParams(use_tc_tiling_on_sc=False)


def _nt(total_chunks, nworkers, wid):
  base, extra = divmod(total_chunks, nworkers)
  return jnp.where(wid < extra, base + 1, base)


def _idx_scratch(n):
  return [pltpu.VMEM((128,), jnp.int32) for _ in range(n)]


def _pipe_scratch():
  return ([pltpu.VMEM((CHUNK,), jnp.int32) for _ in range(2)]
          + [pltpu.VMEM((128,), jnp.int32) for _ in range(2 * CHUNK_ROWS)]
          + [pltpu.VMEM((CHUNK, HID), F32) for _ in range(2)])


def _unpack_pipe(rest):
  src = rest[:2]
  dst = rest[2:2 + 2 * CHUNK_ROWS]
  rows = rest[2 + 2 * CHUNK_ROWS:4 + 2 * CHUNK_ROWS]
  bufs = [(src[0], list(dst[:CHUNK_ROWS]), rows[0]),
          (src[1], list(dst[CHUNK_ROWS:]), rows[1])]
  return bufs, rest[4 + 2 * CHUNK_ROWS:]


def _remap(idl, lo, rlen):
  for g in range(CHUNK_ROWS):
    def kbody(k, _, g=g):
      dv = idl[g][pl.ds(k * 16, 16)]
      inr = (dv >= lo) & (dv < lo + rlen)
      idl[g][pl.ds(k * 16, 16)] = jnp.where(inr, dv - lo, rlen)
      return 0
    lax.fori_loop(0, 128 // 16, kbody, 0)


def _sum_rel(es, ed, h, acc, bufs, sems, nt, first, stride,
             lo=None, rlen=None):
  sem_ld, sem_g, sem_s = sems

  def load(b, j):
    descs = [pltpu.async_copy(es.at[pl.ds(j * CHUNK, CHUNK)],
                              bufs[b][0], sem_ld)]
    descs += [
        pltpu.async_copy(ed.at[pl.ds(j * CHUNK + g * 128, 128)],
                         bufs[b][1][g], sem_ld)
        for g in range(CHUNK_ROWS)
    ]
    return descs

  def gather(b):
    return [
        pltpu.async_copy(h.at[bufs[b][0].at[pl.ds(g * 128, 128)]],
                         bufs[b][2].at[pl.ds(g * 128, 128)], sem_g[b])
        for g in range(CHUNK_ROWS)
    ]

  def scatter(b):
    for g in range(CHUNK_ROWS):
      pltpu.async_copy(bufs[b][2].at[pl.ds(g * 128, 128)],
                       acc.at[bufs[b][1][g]], sem_s[b], add=True)

  def drain_scatter(b):
    pltpu.make_async_copy(h.at[pl.ds(0, CHUNK)], bufs[b][2], sem_s[b]).wait()

  def pair(t2, _):
    @pl.when(t2 > 0)
    def _():
      drain_scatter(0)
      drain_scatter(1)
    j0 = first + (2 * t2) * stride
    lds = load(0, j0) + load(1, j0 + stride)
    for d in lds:
      d.wait()
    g0 = gather(0)
    if lo is not None:
      _remap(bufs[0][1], lo, rlen)
    for d in g0:
      d.wait()
    scatter(0)
    g1 = gather(1)
    if lo is not None:
      _remap(bufs[1][1], lo, rlen)
    for d in g1:
      d.wait()
    scatter(1)
    return 0

  npairs = nt // 2
  lax.fori_loop(0, npairs, pair, 0)
  @pl.when(npairs > 0)
  def _():
    drain_scatter(0)
    drain_scatter(1)

  @pl.when(nt % 2 == 1)
  def _():
    j = first + (nt - 1) * stride
    lds = load(0, j)
    for d in lds:
      d.wait()
    g0 = gather(0)
    if lo is not None:
      _remap(bufs[0][1], lo, rlen)
    for d in g0:
      d.wait()
    scatter(0)
    drain_scatter(0)


def _count_rel(ed, csh, idl, obuf, o16_hbm, sem_ld, sem_s, nt, first, stride,
               lo=None, rlen=None):
  def tbody(t, _):
    j = first + t * stride
    lds = [
        pltpu.async_copy(ed.at[pl.ds(j * CHUNK + g * 128, 128)], idl[g],
                         sem_ld)
        for g in range(CHUNK_ROWS)
    ]
    for d in lds:
      d.wait()
    if lo is not None:
      _remap(idl, lo, rlen)
    for g in range(CHUNK_ROWS):
      pltpu.async_copy(obuf, csh.at[idl[g]], sem_s, add=True)
    for g in range(CHUNK_ROWS):
      pltpu.make_async_copy(o16_hbm, obuf, sem_s).wait()
    return 0

  lax.fori_loop(0, nt, tbody, 0)


def _sc_counts(ed_ai, ed_di, ed_ww, z16, o16):
  chunks_ai = ed_ai.shape[0] // CHUNK
  chunks_di = ed_di.shape[0] // CHUNK
  chunks_ww = ed_ww.shape[0] // CHUNK

  mesh = plsc.VectorSubcoreMesh(core_axis_name="c", subcore_axis_name="s")

  @functools.partial(
      pl.kernel,
      out_type=(jax.ShapeDtypeStruct((NC, N_MOVIE, 16), F32),
                jax.ShapeDtypeStruct((NC, N_MOVIE, 16), F32),
                jax.ShapeDtypeStruct((NC, W_CNT, 16), F32)),
      mesh=mesh,
      compiler_params=_SC_PARAMS,
      scratch_types=_idx_scratch(CHUNK_ROWS) + [
          pltpu.VMEM((128, 16), F32),
          pltpu.VMEM_SHARED((M_ACC, 16), F32),
          pltpu.VMEM_SHARED((M_ACC, 16), F32),
          pltpu.VMEM_SHARED((W_CNT, 16), F32),
          pltpu.SemaphoreType.DMA,
          pltpu.SemaphoreType.DMA,
      ])
  def body(ed_ai_r, ed_di_r, ed_ww_r, z16_r, o16_r, c_ai_r, c_di_r, c_ww_r,
           *rest):
    idl = list(rest[:CHUNK_ROWS])
    obuf, csh_a, csh_d, csh_w, sem_ld, sem_s = rest[CHUNK_ROWS:]
    c = lax.axis_index("c")
    s = lax.axis_index("s")
    w = c * NS + s

    pltpu.sync_copy(o16_r, obuf)
    rpt = M_ACC // NS
    pltpu.sync_copy(z16_r.at[pl.ds(0, rpt)], csh_a.at[pl.ds(s * rpt, rpt)])
    pltpu.sync_copy(z16_r.at[pl.ds(0, rpt)], csh_d.at[pl.ds(s * rpt, rpt)])
    wpt = W_CNT // NS
    pltpu.sync_copy(z16_r.at[pl.ds(0, 1024)],
                    csh_w.at[pl.ds(s * wpt, 1024)])
    pltpu.sync_copy(z16_r.at[pl.ds(0, wpt - 1024)],
                    csh_w.at[pl.ds(s * wpt + 1024, wpt - 1024)])
    plsc.subcore_barrier()

    _count_rel(ed_ai_r, csh_a, idl, obuf, o16_r, sem_ld, sem_s,
               _nt(chunks_ai, NW, w), w, NW)
    _count_rel(ed_di_r, csh_d, idl, obuf, o16_r, sem_ld, sem_s,
               _nt(chunks_di, NW, w), w, NW)
    _count_rel(ed_ww_r, csh_w, idl, obuf, o16_r, sem_ld, sem_s,
               _nt(chunks_ww, NS, s), s, NS, lo=c * HALF, rlen=HALF)

    plsc.subcore_barrier()
    @pl.when(s < N_MOVIE // 1000)
    def _():
      pltpu.sync_copy(csh_a.at[pl.ds(s * 1000, 1000)],
                      c_ai_r.at[c, pl.ds(s * 1000, 1000)])
      pltpu.sync_copy(csh_d.at[pl.ds(s * 1000, 1000)],
                      c_di_r.at[c, pl.ds(s * 1000, 1000)])
    pltpu.sync_copy(csh_w.at[pl.ds(s * wpt, wpt)],
                    c_ww_r.at[c, pl.ds(s * wpt, wpt)])

  return body(ed_ai, ed_di, ed_ww, z16, o16)


def _sc_rel(h, es, ed, z64, tok):
  chunks = es.shape[0] // CHUNK

  mesh = plsc.VectorSubcoreMesh(core_axis_name="c", subcore_axis_name="s")

  @functools.partial(
      pl.kernel,
      out_type=jax.ShapeDtypeStruct((NC, N_MOVIE, HID), F32),
      mesh=mesh,
      compiler_params=_SC_PARAMS,
      scratch_types=_pipe_scratch() + [
          pltpu.VMEM_SHARED((M_ACC, HID), F32),
          pltpu.SemaphoreType.DMA,
          pltpu.SemaphoreType.DMA,
          pltpu.SemaphoreType.DMA,
      ])
  def body(h_r, es_r, ed_r, z64_r, tok_r, s_r, *rest):
    del tok_r
    bufs, rest = _unpack_pipe(rest)
    acc, sem_ld, sg, ss = rest
    c = lax.axis_index("c")
    s = lax.axis_index("s")
    w = c * NS + s

    rpt = M_ACC // NS
    pltpu.sync_copy(z64_r.at[pl.ds(0, rpt)], acc.at[pl.ds(s * rpt, rpt)])
    plsc.subcore_barrier()

    _sum_rel(es_r, ed_r, h_r, acc, bufs, (sem_ld, [sg, sg], [ss, ss]),
             _nt(chunks, NW, w), w, NW)

    plsc.subcore_barrier()
    @pl.when(s < N_MOVIE // 1000)
    def _():
      pltpu.sync_copy(acc.at[pl.ds(s * 1000, 1000)],
                      s_r.at[c, pl.ds(s * 1000, 1000)])

  return body(h, es, ed, z64, tok)


def _sc_ww_range(h, es, ed, z64, base, tok):
  chunks = es.shape[0] // CHUNK

  mesh = plsc.VectorSubcoreMesh(core_axis_name="c", subcore_axis_name="s")

  @functools.partial(
      pl.kernel,
      out_type=jax.ShapeDtypeStruct((NC, WQ_ACC, HID), F32),
      mesh=mesh,
      compiler_params=_SC_PARAMS,
      scratch_types=_pipe_scratch() + [
          pltpu.VMEM_SHARED((WQ_ACC, HID), F32),
          pltpu.SemaphoreType.DMA,
          pltpu.SemaphoreType.DMA,
          pltpu.SemaphoreType.DMA,
      ])
  def body(h_r, es_r, ed_r, z64_r, tok_r, s_r, *rest):
    del tok_r
    bufs, rest = _unpack_pipe(rest)
    acc, sem_ld, sg, ss = rest
    c = lax.axis_index("c")
    s = lax.axis_index("s")
    lo = c * HALF + base

    rpt = WQ_ACC // NS
    pltpu.sync_copy(z64_r.at[pl.ds(0, rpt)], acc.at[pl.ds(s * rpt, rpt)])
    plsc.subcore_barrier()

    _sum_rel(es_r, ed_r, h_r, acc, bufs, (sem_ld, [sg, sg], [ss, ss]),
             _nt(chunks, NS, s), s, NS, lo=lo, rlen=WQ)

    plsc.subcore_barrier()
    pltpu.sync_copy(acc.at[pl.ds(s * rpt, rpt)],
                    s_r.at[c, pl.ds(s * rpt, rpt)])

  return body(h, es, ed, z64, tok)


def _tc_proj0(x_person, Wl0_ai, Wl0_di, Wl0_ww, Wr0_ww):
  R = 1000
  grid = (N_PERSON // R,)
  D = x_person.shape[1]

  def body(x_r, wa_r, wd_r, ww_r, wr_r, ha_r, hd_r, hw_r, xr_r):
    x = x_r[...]
    ha_r[...] = jnp.dot(x, wa_r[...], preferred_element_type=F32)
    hd_r[...] = jnp.dot(x, wd_r[...], preferred_element_type=F32)
    hw_r[...] = jnp.dot(x, ww_r[...], preferred_element_type=F32)
    xr_r[...] = jnp.dot(x, wr_r[...], preferred_element_type=F32)

  wspec = pl.BlockSpec((D, HID), lambda i: (0, 0))
  ospec = pl.BlockSpec((R, HID), lambda i: (i, 0))
  return pl.pallas_call(
      body,
      grid=grid,
      in_specs=[pl.BlockSpec((R, D), lambda i: (i, 0))] + [wspec] * 4,
      out_specs=[ospec] * 4,
      out_shape=[jax.ShapeDtypeStruct((N_PERSON, HID), F32)] * 4,
  )(x_person, Wl0_ai, Wl0_di, Wl0_ww, Wr0_ww)


def _tc_movie_combine(S_ai, c_ai, S_di, c_di, x_movie, Wr0_ai, Wr0_di,
                      bl0_ai, bl0_di):
  R = 1000
  D = x_movie.shape[1]

  def body(sa_r, ca_r, sd_r, cd_r, x_r, wa_r, wd_r, ba_r, bd_r,
           mv_r, cma_r, cmd_r):
    ca = jnp.maximum(ca_r[0] + ca_r[1], 1.0)
    cd = jnp.maximum(cd_r[0] + cd_r[1], 1.0)
    sa = (sa_r[0] + sa_r[1]) / ca
    sd = (sd_r[0] + sd_r[1]) / cd
    xr = jnp.dot(x_r[...], wa_r[...] + wd_r[...], preferred_element_type=F32)
    mv_r[...] = jnp.maximum(sa + sd + xr + ba_r[...] + bd_r[...], 0.0)
    cma_r[...] = ca
    cmd_r[...] = cd

  sspec = pl.BlockSpec((NC, R, HID), lambda i: (0, i, 0))
  cspec = pl.BlockSpec((NC, R, 1), lambda i: (0, i, 0))
  wspec = pl.BlockSpec((D, HID), lambda i: (0, 0))
  bspec = pl.BlockSpec((HID,), lambda i: (0,))
  return pl.pallas_call(
      body,
      grid=(N_MOVIE // R,),
      in_specs=[sspec, cspec, sspec, cspec,
                pl.BlockSpec((R, D), lambda i: (i, 0)), wspec, wspec,
                bspec, bspec],
      out_specs=[pl.BlockSpec((R, HID), lambda i: (i, 0)),
                 pl.BlockSpec((R, 1), lambda i: (i, 0)),
                 pl.BlockSpec((R, 1), lambda i: (i, 0))],
      out_shape=[jax.ShapeDtypeStruct((N_MOVIE, HID), F32),
                 jax.ShapeDtypeStruct((N_MOVIE, 1), F32),
                 jax.ShapeDtypeStruct((N_MOVIE, 1), F32)],
  )(S_ai, c_ai, S_di, c_di, x_movie, Wr0_ai, Wr0_di, bl0_ai, bl0_di)


def _tc_pe_proj1(S_ww, cw_col, xr_ww, bl0_ww, Wl1_ai, Wl1_di):
  R = 1000
  grid = (N_PERSON // R,)

  def body(sw_r, cw_r, xr_r, b_r, wa_r, wd_r, ga_r, gd_r):
    cw = jnp.maximum(cw_r[...], 1.0)
    pe = jnp.maximum(sw_r[...] / cw + xr_r[...] + b_r[...], 0.0)
    ga_r[...] = jnp.dot(pe, wa_r[...], preferred_element_type=F32)
    gd_r[...] = jnp.dot(pe, wd_r[...], preferred_element_type=F32)

  ospec = pl.BlockSpec((R, HID), lambda i: (i, 0))
  return pl.pallas_call(
      body,
      grid=grid,
      in_specs=[
          pl.BlockSpec((R, HID), lambda i: (i, 0)),
          pl.BlockSpec((R, 1), lambda i: (i, 0)),
          pl.BlockSpec((R, HID), lambda i: (i, 0)),
          pl.BlockSpec((HID,), lambda i: (0,)),
          pl.BlockSpec((HID, HID), lambda i: (0, 0)),
          pl.BlockSpec((HID, HID), lambda i: (0, 0)),
      ],
      out_specs=[ospec, ospec],
      out_shape=[jax.ShapeDtypeStruct((N_PERSON, HID), F32)] * 2,
  )(S_ww, cw_col, xr_ww, bl0_ww, Wl1_ai, Wl1_di)


def _tc_head(S1_ai, S1_di, cma, cmd, mv, Wr1_ai, Wr1_di, bl1_ai, bl1_di,
             w_out_row, b_out):
  R = 1000

  def body(sa_r, sd_r, ca_r, cd_r, mv_r, wa_r, wd_r, ba_r, bd_r, wo_r, bo_r,
           out_r):
    xr = jnp.dot(mv_r[...], wa_r[...] + wd_r[...], preferred_element_type=F32)
    mv2 = jnp.maximum(
        (sa_r[0] + sa_r[1]) / ca_r[...] + (sd_r[0] + sd_r[1]) / cd_r[...]
        + xr + ba_r[...] + bd_r[...], 0.0)
    out_r[...] = jnp.dot(mv2, wo_r[...], preferred_element_type=F32) + bo_r[0]

  sspec = pl.BlockSpec((NC, R, HID), lambda i: (0, i, 0))
  cspec = pl.BlockSpec((R, 1), lambda i: (i, 0))
  wspec = pl.BlockSpec((HID, HID), lambda i: (0, 0))
  bspec = pl.BlockSpec((HID,), lambda i: (0,))
  return pl.pallas_call(
      body,
      grid=(N_MOVIE // R,),
      in_specs=[sspec, sspec, cspec, cspec,
                pl.BlockSpec((R, HID), lambda i: (i, 0)), wspec, wspec,
                bspec, bspec,
                pl.BlockSpec((HID, 1), lambda i: (0, 0)),
                pl.BlockSpec((1,), lambda i: (0,))],
      out_specs=pl.BlockSpec((R, 1), lambda i: (i, 0)),
      out_shape=jax.ShapeDtypeStruct((N_MOVIE, 1), F32),
  )(S1_ai, S1_di, cma, cmd, mv, Wr1_ai, Wr1_di, bl1_ai, bl1_di,
    w_out_row, b_out)


def kernel(x_person, x_movie, e_ai_src, e_ai_dst, e_di_src, e_di_dst,
           e_ww_src, e_ww_dst, Wl0_ai, bl0_ai, Wr0_ai, Wl0_di, bl0_di, Wr0_di,
           Wl0_ww, bl0_ww, Wr0_ww, Wl1_ai, bl1_ai, Wr1_ai, Wl1_di, bl1_di,
           Wr1_di, Wl1_ww, bl1_ww, Wr1_ww, W_out, b_out):
  def _prep(src, dst, pad_dst):
    n = src.shape[0]
    m = -(-n // CHUNK) * CHUNK
    es = jnp.pad(src.astype(jnp.int32), (0, m - n))
    ed = jnp.pad(dst.astype(jnp.int32), (0, m - n), constant_values=pad_dst)
    return es, ed

  es_ai, ed_ai = _prep(e_ai_src, e_ai_dst, N_MOVIE)
  es_di, ed_di = _prep(e_di_src, e_di_dst, N_MOVIE)
  es_ww, ed_ww = _prep(e_ww_src, e_ww_dst, -1)

  z64 = jnp.zeros((1024, HID), F32)
  z16 = jnp.zeros((1024, 16), F32)
  o16 = jnp.ones((128, 16), F32)

  c_ai_raw, c_di_raw, c_ww_raw = _sc_counts(ed_ai, ed_di, ed_ww, z16, o16)

  h_ai, h_di, h_ww, xr_ww = _tc_proj0(x_person, Wl0_ai, Wl0_di, Wl0_ww, Wr0_ww)

  S_ai = _sc_rel(h_ai, es_ai, ed_ai, z64, c_ai_raw[0, :8])
  S_di = _sc_rel(h_di, es_di, ed_di, z64, S_ai[0, :8, :16])
  Sw0 = _sc_ww_range(h_ww, es_ww, ed_ww, z64, 0, S_di[0, :8, :16])
  Sw1 = _sc_ww_range(h_ww, es_ww, ed_ww, z64, WQ, Sw0[0, :8, :16])
  S_ww = jnp.concatenate(
      [Sw0[0, :WQ], Sw1[0, :WQ], Sw0[1, :WQ], Sw1[1, :WQ]], axis=0)

  c_ai = c_ai_raw[:, :, :1]
  c_di = c_di_raw[:, :, :1]
  cw_col = jnp.concatenate([c_ww_raw[0, :HALF], c_ww_raw[1, :HALF]],
                           axis=0)[:, :1]

  mv, cma, cmd = _tc_movie_combine(S_ai, c_ai, S_di, c_di, x_movie,
                                   Wr0_ai, Wr0_di, bl0_ai, bl0_di)
  g_ai, g_di = _tc_pe_proj1(S_ww, cw_col, xr_ww, bl0_ww, Wl1_ai, Wl1_di)

  S1_ai = _sc_rel(g_ai, es_ai, ed_ai, z64, Sw1[0, :8, :16])
  S1_di = _sc_rel(g_di, es_di, ed_di, z64, S1_ai[0, :8, :16])

  out = _tc_head(S1_ai, S1_di, cma, cmd, mv, Wr1_ai, Wr1_di, bl1_ai, bl1_di,
                 W_out, b_out)
  return out.reshape(N_MOVIE)
